# baseline (speedup 1.0000x reference)
import functools

import jax
import jax.numpy as jnp
from jax import lax
from jax.experimental import pallas as pl
from jax.experimental.pallas import tpu as pltpu
from jax.experimental.pallas import tpu_sc as plsc

_SCALE = 0.707106781
_NC = 2
_NS = 16
_NW = _NC * _NS
_WC = 128
_LANES = 16
_SC_ROWS = 192


def _sc_body(h_hbm, w_hbm, out_hbm, h_v, w_v, ob0, ob1, sem0, sem1):
    max_w, dim = w_hbm.shape
    sc_rows = h_hbm.shape[0] // dim
    nvd = dim // _LANES
    rows_per = sc_rows // _NW
    n_chunks = max_w // _WC

    c = lax.axis_index("c")
    s = lax.axis_index("s")
    wid = s * _NC + c
    base_h = wid * rows_per

    pltpu.sync_copy(
        h_hbm.at[pl.ds(pl.multiple_of(base_h * dim, 128), rows_per * dim)], h_v
    )

    def compute_tile(h, ob):
        hr = [
            h_v[pl.ds(pl.multiple_of(h * dim + _LANES * d, _LANES), _LANES)]
            * _SCALE
            for d in range(nvd)
        ]

        def w_body(wi, _):
            for d in range(nvd):
                sl = pl.ds(_LANES * d, _LANES)
                ob[wi, sl] = w_v[wi, sl] * _SCALE + hr[d]
            return 0

        lax.fori_loop(0, _WC, w_body, 0, unroll=2)

    for wc in range(n_chunks):
        pltpu.sync_copy(w_hbm.at[pl.ds(wc * _WC, _WC)], w_v)

        def pair_body(p, _, wc=wc):
            h0 = 2 * p
            h1 = 2 * p + 1

            @pl.when(p > 0)
            def _():
                pltpu.make_async_copy(
                    ob0, out_hbm.at[base_h, pl.ds(wc * _WC, _WC), :], sem0
                ).wait()

            compute_tile(h0, ob0)
            pltpu.async_copy(
                ob0, out_hbm.at[base_h + h0, pl.ds(wc * _WC, _WC), :], sem0
            )

            @pl.when(p > 0)
            def _():
                pltpu.make_async_copy(
                    ob1, out_hbm.at[base_h, pl.ds(wc * _WC, _WC), :], sem1
                ).wait()

            compute_tile(h1, ob1)
            pltpu.async_copy(
                ob1, out_hbm.at[base_h + h1, pl.ds(wc * _WC, _WC), :], sem1
            )
            return 0

        lax.fori_loop(0, rows_per // 2, pair_body, 0)

        pltpu.make_async_copy(
            ob0, out_hbm.at[base_h, pl.ds(wc * _WC, _WC), :], sem0
        ).wait()
        pltpu.make_async_copy(
            ob1, out_hbm.at[base_h, pl.ds(wc * _WC, _WC), :], sem1
        ).wait()


def _sc_part(h_rows, w_embed):
    sc_rows, dim = h_rows.shape
    max_w = w_embed.shape[0]
    mesh = plsc.VectorSubcoreMesh(core_axis_name="c", subcore_axis_name="s")
    k = functools.partial(
        pl.kernel,
        mesh=mesh,
        out_type=jax.ShapeDtypeStruct((sc_rows, max_w, dim), jnp.float32),
        scratch_types=[
            pltpu.VMEM((sc_rows // _NW * dim,), jnp.float32),
            pltpu.VMEM((_WC, dim), jnp.float32),
            pltpu.VMEM((_WC, dim), jnp.float32),
            pltpu.VMEM((_WC, dim), jnp.float32),
            pltpu.SemaphoreType.DMA,
            pltpu.SemaphoreType.DMA,
        ],
    )(_sc_body)
    return k(h_rows.reshape(-1), w_embed)


def _tc_body(h_ref, w_ref, o_ref):
    hs = h_ref[...] * _SCALE
    ws = w_ref[...] * _SCALE
    o_ref[...] = hs[:, None, :] + ws[None, :, :]


def _tc_part(h_rows, w_embed):
    tc_rows, dim = h_rows.shape
    max_w = w_embed.shape[0]
    bh = 8
    return pl.pallas_call(
        _tc_body,
        grid=(tc_rows // bh,),
        in_specs=[
            pl.BlockSpec((bh, dim), lambda i: (i, 0)),
            pl.BlockSpec((max_w, dim), lambda i: (0, 0)),
        ],
        out_specs=pl.BlockSpec((bh, max_w, dim), lambda i: (i, 0, 0)),
        out_shape=jax.ShapeDtypeStruct((tc_rows, max_w, dim), jnp.float32),
    )(h_rows, w_embed)


def kernel(height, width, h_embed, w_embed):
    max_h = h_embed.shape[0]
    tc_rows = max_h - _SC_ROWS
    sc_out = _sc_part(h_embed[tc_rows:], w_embed)
    tc_out = _tc_part(h_embed[:tc_rows], w_embed)
    return jnp.concatenate([tc_out, sc_out], axis=0)


# speedup vs baseline: 3.4221x; 3.4221x over previous
import jax
import jax.numpy as jnp
from jax.experimental import pallas as pl

_SCALE = 0.707106781


def _body(h_ref, w_ref, o_ref):
    hs = h_ref[...] * _SCALE
    ws = w_ref[...] * _SCALE
    o_ref[...] = hs[:, None, :] + ws[None, :, :]


def kernel(height, width, h_embed, w_embed):
    max_h, dim = h_embed.shape
    max_w = w_embed.shape[0]
    bh = 32
    return pl.pallas_call(
        _body,
        grid=(max_h // bh,),
        in_specs=[
            pl.BlockSpec((bh, dim), lambda i: (i, 0)),
            pl.BlockSpec((max_w, dim), lambda i: (0, 0)),
        ],
        out_specs=pl.BlockSpec((bh, max_w, dim), lambda i: (i, 0, 0)),
        out_shape=jax.ShapeDtypeStruct((max_h, max_w, dim), jnp.float32),
    )(h_embed, w_embed)
